# Initial kernel scaffold; baseline (speedup 1.0000x reference)
import jax
import jax.numpy as jnp
from jax.experimental import pallas as pl


def kernel(pos, normal, W1a, b1a, g1, be1, W1b, b1b, W2a, b2a, g2, be2, W2b, b2b, W3a, b3a, g3, be3, W3b, b3b):
    raise NotImplementedError("write your pallas kernel here")


# speedup vs baseline: 1.2127x; 1.2127x over previous
import functools

import jax
import jax.numpy as jnp
from jax import lax
from jax.experimental import pallas as pl
from jax.experimental.pallas import tpu as pltpu

N = 10000
K = 16
_EPS = 1e-5


def _ab_body(x_ref, p_ref, wt_ref, wb_ref, a_ref, b_ref):
    bvec = jnp.dot(p_ref[...], wb_ref[...], preferred_element_type=jnp.float32)
    a_ref[...] = jnp.dot(x_ref[...], wt_ref[...], preferred_element_type=jnp.float32) + bvec
    b_ref[...] = bvec


def _node_ab(x, pos, Wa):
    fin = x.shape[1]
    C = Wa.shape[1]
    wt = Wa[:fin]
    wb = Wa[fin:]
    return pl.pallas_call(
        _ab_body,
        out_shape=(
            jax.ShapeDtypeStruct((N, C), jnp.float32),
            jax.ShapeDtypeStruct((N, C), jnp.float32),
        ),
    )(x, pos, wt, wb)


def _conv_body(g_ref, b_ref, wb_ref, s_ref, prm_ref, o_ref):
    Kc, P, C = g_ref.shape
    ba = prm_ref[0:1, :]
    gm = prm_ref[1:2, :]
    bt = prm_ref[2:3, :]
    bb = prm_ref[3:4, :]
    s = s_ref[...]
    h = (g_ref[...] - b_ref[...][None]).reshape(Kc * P, C) + ba
    m = jnp.dot(h, s, preferred_element_type=jnp.float32)
    d = h - m
    v = jnp.dot(d * d, s, preferred_element_type=jnp.float32)
    hn = (d / jnp.sqrt(v + _EPS)) * gm + bt
    hr = jnp.maximum(hn, 0.0)
    z = jnp.dot(hr, wb_ref[...], preferred_element_type=jnp.float32)
    acc = jnp.max(z.reshape(Kc, P, C), axis=0)
    o_ref[...] = jnp.maximum(acc + bb, 0.0)


def _conv_layer(G, B, Wb, prm, S, P=400):
    C = B.shape[1]
    grid = (N // P,)
    return pl.pallas_call(
        _conv_body,
        grid=grid,
        in_specs=[
            pl.BlockSpec((K, P, C), lambda i: (0, i, 0)),
            pl.BlockSpec((P, C), lambda i: (i, 0)),
            pl.BlockSpec((C, C), lambda i: (0, 0)),
            pl.BlockSpec((C, C), lambda i: (0, 0)),
            pl.BlockSpec((8, C), lambda i: (0, 0)),
        ],
        out_specs=pl.BlockSpec((P, C), lambda i: (i, 0)),
        out_shape=jax.ShapeDtypeStruct((N, C), jnp.float32),
    )(G, B, Wb, S, prm)


def _group_avg_matrix(C):
    i = jnp.arange(C)
    return jnp.where((i[:, None] // 8) == (i[None, :] // 8), 1.0 / 8.0, 0.0).astype(jnp.float32)


def _pack_params(ba, gm, bt, bb):
    C = ba.shape[0]
    p = jnp.zeros((8, C), jnp.float32)
    return p.at[0].set(ba).at[1].set(gm).at[2].set(bt).at[3].set(bb)


def kernel(pos, normal, W1a, b1a, g1, be1, W1b, b1b, W2a, b2a, g2, be2, W2b, b2b,
           W3a, b3a, g3, be3, W3b, b3b):
    sq = jnp.sum(pos * pos, axis=1)
    d = sq[None, :] - 2.0 * (pos @ pos.T) + sq[:, None]
    _, nbr = jax.lax.top_k(-d, K)
    nbr_t = nbr.T

    x0 = jnp.concatenate([pos, normal], axis=-1)

    def layer(x, Wa, ba, gm, bt, Wb, bb):
        A, B = _node_ab(x, pos, Wa)
        G = A[nbr_t]
        C = Wa.shape[1]
        return _conv_layer(G, B, Wb, _pack_params(ba, gm, bt, bb), _group_avg_matrix(C))

    h1 = layer(x0, W1a, b1a, g1, be1, W1b, b1b)
    h2 = layer(h1, W2a, b2a, g2, be2, W2b, b2b)
    h3 = layer(h2, W3a, b3a, g3, be3, W3b, b3b)
    return (h1, h2, h3)
